# Initial kernel scaffold; baseline (speedup 1.0000x reference)
import jax
import jax.numpy as jnp
from jax.experimental import pallas as pl


def kernel(x, W_enc, b_enc, W_gate, W_experts, b_experts, W_res, b_res, W_coef, b_coef, W_dec, b_dec):
    raise NotImplementedError("write your pallas kernel here")


# speedup vs baseline: 3.2601x; 3.2601x over previous
import functools

import jax
import jax.numpy as jnp
from jax import lax
from jax.experimental import pallas as pl
from jax.experimental.pallas import tpu as pltpu
from jax.experimental.pallas import tpu_sc as plsc

D_IN = 1024
D_H = 512
N_EXP = 64
N_TOK = 4096
TILE = 128
N_TILES = N_TOK // TILE


def _stage1_body(x_ref, we_ref, be_ref, wg_ref, wr_ref, br_ref, wc_ref, bc_ref,
                 h_ref, rc_ref, g_ref, idx_ref, rank_ref, cnt_ref, offs_ref,
                 acc):
    t = pl.program_id(0)

    @pl.when(t == 0)
    def _():
        acc[...] = jnp.zeros_like(acc)

    x = x_ref[...]
    h = jnp.maximum(
        jnp.dot(x, we_ref[...], preferred_element_type=jnp.float32)
        + be_ref[...], 0.0)
    h_ref[...] = h

    logits = jnp.dot(h, wg_ref[...], preferred_element_type=jnp.float32)
    mx = jnp.max(logits, axis=1, keepdims=True)
    ssum = jnp.sum(jnp.exp(logits - mx), axis=1, keepdims=True)
    gate_val = 1.0 / ssum
    idx = jnp.argmax(logits, axis=1).astype(jnp.int32)

    cols = lax.broadcasted_iota(jnp.int32, (TILE, N_EXP), 1)
    m = (cols == idx[:, None]).astype(jnp.float32)
    rows_i = lax.broadcasted_iota(jnp.int32, (TILE, TILE), 0)
    cols_i = lax.broadcasted_iota(jnp.int32, (TILE, TILE), 1)
    ltri = (rows_i > cols_i).astype(jnp.float32)
    rank_tile = jnp.dot(ltri, m, preferred_element_type=jnp.float32)
    rank = jnp.sum(m * (rank_tile + acc[...]), axis=1, keepdims=True)
    rank_ref[...] = rank.astype(jnp.int32)
    idx_ref[...] = idx[:, None]
    acc[...] = acc[...] + jnp.sum(m, axis=0, keepdims=True)

    res = jnp.dot(h, wr_ref[...], preferred_element_type=jnp.float32) + br_ref[...]
    cl = jnp.dot(h, wc_ref[...], preferred_element_type=jnp.float32) + bc_ref[...]
    cmx = jnp.max(cl, axis=1, keepdims=True)
    ce = jnp.exp(cl - cmx)
    cs = jnp.sum(ce, axis=1, keepdims=True)
    rc_ref[...] = res * (ce[:, 1:2] / cs)
    g_ref[...] = gate_val * (ce[:, 0:1] / cs)

    @pl.when(t == N_TILES - 1)
    def _():
        cnt = acc[...]
        ea = lax.broadcasted_iota(jnp.int32, (N_EXP, N_EXP), 0)
        eb = lax.broadcasted_iota(jnp.int32, (N_EXP, N_EXP), 1)
        ustri = (ea < eb).astype(jnp.float32)
        offs = jnp.dot(cnt, ustri, preferred_element_type=jnp.float32)
        cnt_ref[...] = cnt.astype(jnp.int32)
        offs_ref[...] = offs.astype(jnp.int32)


def _stage1(x, w_enc, b_enc, w_gate, w_res, b_res, w_coef, b_coef):
    out_shapes = (
        jax.ShapeDtypeStruct((N_TOK, D_H), jnp.float32),
        jax.ShapeDtypeStruct((N_TOK, D_H), jnp.float32),
        jax.ShapeDtypeStruct((N_TOK, 1), jnp.float32),
        jax.ShapeDtypeStruct((N_TOK, 1), jnp.int32),
        jax.ShapeDtypeStruct((N_TOK, 1), jnp.int32),
        jax.ShapeDtypeStruct((1, N_EXP), jnp.int32),
        jax.ShapeDtypeStruct((1, N_EXP), jnp.int32),
    )
    full = lambda shape: pl.BlockSpec(shape, lambda t: (0, 0))
    tok = lambda w: pl.BlockSpec((TILE, w), lambda t: (t, 0))
    return pl.pallas_call(
        _stage1_body,
        grid=(N_TILES,),
        in_specs=[
            tok(D_IN),
            full((D_IN, D_H)),
            full((1, D_H)),
            full((D_H, N_EXP)),
            full((D_H, D_H)),
            full((1, D_H)),
            full((D_H, 2)),
            full((1, 2)),
        ],
        out_specs=[
            tok(D_H), tok(D_H), tok(1), tok(1), tok(1),
            full((1, N_EXP)), full((1, N_EXP)),
        ],
        out_shape=out_shapes,
        scratch_shapes=[pltpu.VMEM((1, N_EXP), jnp.float32)],
    )(x, w_enc, b_enc, w_gate, w_res, b_res, w_coef, b_coef)


_NC = 2
_NS = 16
_NW = _NC * _NS
_CHUNK = N_TOK // _NW
_LANES = 16


def _sc_dispatch(h, idx, rank, offs):
    mesh = plsc.VectorSubcoreMesh(core_axis_name="c", subcore_axis_name="s")

    @functools.partial(
        pl.kernel, mesh=mesh,
        out_type=[
            jax.ShapeDtypeStruct((N_TOK, D_H), jnp.float32),
            jax.ShapeDtypeStruct((N_TOK,), jnp.int32),
        ],
        scratch_types=[
            pltpu.VMEM((_CHUNK,), jnp.int32),
            pltpu.VMEM((_CHUNK,), jnp.int32),
            pltpu.VMEM((_CHUNK,), jnp.int32),
            pltpu.VMEM((_CHUNK,), jnp.int32),
            pltpu.VMEM((_CHUNK, D_H), jnp.float32),
            pltpu.SemaphoreType.DMA,
        ],
    )
    def k(h_hbm, idx_hbm, rank_hbm, offs_hbm, hs_hbm, pos_hbm,
          idx_v, rank_v, offsg_v, pos_v, rows_v, sem):
        wid = lax.axis_index("s") * _NC + lax.axis_index("c")
        base = wid * _CHUNK
        pltpu.sync_copy(idx_hbm.at[pl.ds(base, _CHUNK)], idx_v)
        pltpu.sync_copy(rank_hbm.at[pl.ds(base, _CHUNK)], rank_v)
        pltpu.async_copy(offs_hbm.at[idx_v], offsg_v, sem).wait()
        for i in range(_CHUNK // _LANES):
            sl = pl.ds(i * _LANES, _LANES)
            pos_v[sl] = offsg_v[sl] + rank_v[sl]
        pltpu.sync_copy(h_hbm.at[pl.ds(base, _CHUNK)], rows_v)
        pltpu.async_copy(rows_v, hs_hbm.at[pos_v], sem).wait()
        pltpu.sync_copy(pos_v, pos_hbm.at[pl.ds(base, _CHUNK)])

    return k(h, idx, rank, offs)


def _sc_combine(ys, pos):
    mesh = plsc.VectorSubcoreMesh(core_axis_name="c", subcore_axis_name="s")

    @functools.partial(
        pl.kernel, mesh=mesh,
        out_type=jax.ShapeDtypeStruct((N_TOK, D_H), jnp.float32),
        scratch_types=[
            pltpu.VMEM((_CHUNK,), jnp.int32),
            pltpu.VMEM((_CHUNK, D_H), jnp.float32),
            pltpu.SemaphoreType.DMA,
        ],
    )
    def k(ys_hbm, pos_hbm, out_hbm, pos_v, rows_v, sem):
        wid = lax.axis_index("s") * _NC + lax.axis_index("c")
        base = wid * _CHUNK
        pltpu.sync_copy(pos_hbm.at[pl.ds(base, _CHUNK)], pos_v)
        pltpu.async_copy(ys_hbm.at[pos_v], rows_v, sem).wait()
        pltpu.sync_copy(rows_v, out_hbm.at[pl.ds(base, _CHUNK)])

    return k(ys, pos)


def _stage2_body(offs_ref, cnt_ref, hs_ref, w_ref, b_ref, out_ref):
    e = pl.program_id(0)
    start = offs_ref[e]
    cnt = cnt_ref[e]
    s0 = (start // 8) * 8
    n = (cnt + (start - s0) + TILE - 1) // TILE
    w = w_ref[0]
    b = b_ref[0]

    def body(j, _):
        s = jnp.minimum(s0 + j * TILE, N_TOK - TILE)
        s = pl.multiple_of(s, 8)
        rows = hs_ref[pl.ds(s, TILE), :]
        y = jnp.dot(rows, w, preferred_element_type=jnp.float32) + b
        rid = s + lax.broadcasted_iota(jnp.int32, (TILE, 1), 0)
        msk = (rid >= start) & (rid < start + cnt)
        cur = out_ref[pl.ds(s, TILE), :]
        out_ref[pl.ds(s, TILE), :] = jnp.where(msk, y, cur)
        return 0

    lax.fori_loop(0, n, body, 0)


def _stage2(offs, cnt, hs, w_experts, b_experts):
    return pl.pallas_call(
        _stage2_body,
        grid=(N_EXP,),
        in_specs=[
            pl.BlockSpec(memory_space=pltpu.SMEM),
            pl.BlockSpec(memory_space=pltpu.SMEM),
            pl.BlockSpec((N_TOK, D_H), lambda e: (0, 0)),
            pl.BlockSpec((1, D_H, D_H), lambda e: (e, 0, 0)),
            pl.BlockSpec((1, 1, D_H), lambda e: (e, 0, 0)),
        ],
        out_specs=pl.BlockSpec((N_TOK, D_H), lambda e: (0, 0)),
        out_shape=jax.ShapeDtypeStruct((N_TOK, D_H), jnp.float32),
    )(offs, cnt, hs, w_experts, b_experts)


def _stage3_body(my_ref, g_ref, rc_ref, wd_ref, bd_ref, out_ref):
    mixed = my_ref[...] * g_ref[...] + rc_ref[...]
    out_ref[...] = (
        jnp.dot(mixed, wd_ref[...], preferred_element_type=jnp.float32)
        + bd_ref[...])


def _stage3(my, g, rc, w_dec, b_dec):
    full = lambda shape: pl.BlockSpec(shape, lambda t: (0, 0))
    tok = lambda w: pl.BlockSpec((TILE, w), lambda t: (t, 0))
    return pl.pallas_call(
        _stage3_body,
        grid=(N_TILES,),
        in_specs=[tok(D_H), tok(1), tok(D_H), full((D_H, D_IN)), full((1, D_IN))],
        out_specs=tok(D_IN),
        out_shape=jax.ShapeDtypeStruct((N_TOK, D_IN), jnp.float32),
    )(my, g, rc, w_dec, b_dec)


def kernel(x, W_enc, b_enc, W_gate, W_experts, b_experts, W_res, b_res,
           W_coef, b_coef, W_dec, b_dec):
    h, rc, g, idx, rank, cnt, offs = _stage1(
        x, W_enc, b_enc.reshape(1, D_H), W_gate, W_res, b_res.reshape(1, D_H),
        W_coef, b_coef.reshape(1, 2))
    hs, pos = _sc_dispatch(h, idx.reshape(N_TOK), rank.reshape(N_TOK),
                           offs.reshape(N_EXP))
    ys = _stage2(offs.reshape(N_EXP), cnt.reshape(N_EXP), hs,
                 W_experts, b_experts.reshape(N_EXP, 1, D_H))
    my = _sc_combine(ys, pos)
    return _stage3(my, g, rc, W_dec, b_dec.reshape(1, D_IN))


# speedup vs baseline: 3.3059x; 1.0141x over previous
import functools

import jax
import jax.numpy as jnp
from jax import lax
from jax.experimental import pallas as pl
from jax.experimental.pallas import tpu as pltpu
from jax.experimental.pallas import tpu_sc as plsc

D_IN = 1024
D_H = 512
N_EXP = 64
N_TOK = 4096
TILE = 128
N_TILES = N_TOK // TILE


def _stage1_body(x_ref, we_ref, be_ref, wg_ref, wr_ref, br_ref, wc_ref, bc_ref,
                 h_ref, rc_ref, g_ref, idx_ref, rank_ref, cnt_ref, offs_ref,
                 acc):
    t = pl.program_id(0)

    @pl.when(t == 0)
    def _():
        acc[...] = jnp.zeros_like(acc)

    x = x_ref[...]
    h = jnp.maximum(
        jnp.dot(x, we_ref[...], preferred_element_type=jnp.float32)
        + be_ref[...], 0.0)
    h_ref[...] = h

    logits = jnp.dot(h, wg_ref[...], preferred_element_type=jnp.float32)
    mx = jnp.max(logits, axis=1, keepdims=True)
    ssum = jnp.sum(jnp.exp(logits - mx), axis=1, keepdims=True)
    gate_val = 1.0 / ssum
    idx = jnp.argmax(logits, axis=1).astype(jnp.int32)

    cols = lax.broadcasted_iota(jnp.int32, (TILE, N_EXP), 1)
    m = (cols == idx[:, None]).astype(jnp.float32)
    rows_i = lax.broadcasted_iota(jnp.int32, (TILE, TILE), 0)
    cols_i = lax.broadcasted_iota(jnp.int32, (TILE, TILE), 1)
    ltri = (rows_i > cols_i).astype(jnp.float32)
    rank_tile = jnp.dot(ltri, m, preferred_element_type=jnp.float32)
    rank = jnp.sum(m * (rank_tile + acc[...]), axis=1, keepdims=True)
    rank_ref[...] = rank.astype(jnp.int32)
    idx_ref[...] = idx[:, None]
    acc[...] = acc[...] + jnp.sum(m, axis=0, keepdims=True)

    res = jnp.dot(h, wr_ref[...], preferred_element_type=jnp.float32) + br_ref[...]
    cl = jnp.dot(h, wc_ref[...], preferred_element_type=jnp.float32) + bc_ref[...]
    cmx = jnp.max(cl, axis=1, keepdims=True)
    ce = jnp.exp(cl - cmx)
    cs = jnp.sum(ce, axis=1, keepdims=True)
    rc_ref[...] = res * (ce[:, 1:2] / cs)
    g_ref[...] = gate_val * (ce[:, 0:1] / cs)

    @pl.when(t == N_TILES - 1)
    def _():
        cnt = acc[...]
        ea = lax.broadcasted_iota(jnp.int32, (N_EXP, N_EXP), 0)
        eb = lax.broadcasted_iota(jnp.int32, (N_EXP, N_EXP), 1)
        ustri = (ea < eb).astype(jnp.float32)
        offs = jnp.dot(cnt, ustri, preferred_element_type=jnp.float32)
        cnt_ref[...] = cnt.astype(jnp.int32)
        offs_ref[...] = offs.astype(jnp.int32)


def _stage1(x, w_enc, b_enc, w_gate, w_res, b_res, w_coef, b_coef):
    out_shapes = (
        jax.ShapeDtypeStruct((N_TOK, D_H), jnp.float32),
        jax.ShapeDtypeStruct((N_TOK, D_H), jnp.float32),
        jax.ShapeDtypeStruct((N_TOK, 1), jnp.float32),
        jax.ShapeDtypeStruct((N_TOK, 1), jnp.int32),
        jax.ShapeDtypeStruct((N_TOK, 1), jnp.int32),
        jax.ShapeDtypeStruct((1, N_EXP), jnp.int32),
        jax.ShapeDtypeStruct((1, N_EXP), jnp.int32),
    )
    full = lambda shape: pl.BlockSpec(shape, lambda t: (0, 0))
    tok = lambda w: pl.BlockSpec((TILE, w), lambda t: (t, 0))
    return pl.pallas_call(
        _stage1_body,
        grid=(N_TILES,),
        in_specs=[
            tok(D_IN),
            full((D_IN, D_H)),
            full((1, D_H)),
            full((D_H, N_EXP)),
            full((D_H, D_H)),
            full((1, D_H)),
            full((D_H, 2)),
            full((1, 2)),
        ],
        out_specs=[
            tok(D_H), tok(D_H), tok(1), tok(1), tok(1),
            full((1, N_EXP)), full((1, N_EXP)),
        ],
        out_shape=out_shapes,
        scratch_shapes=[pltpu.VMEM((1, N_EXP), jnp.float32)],
    )(x, w_enc, b_enc, w_gate, w_res, b_res, w_coef, b_coef)


_NC = 2
_NS = 16
_NW = _NC * _NS
_CHUNK = N_TOK // _NW
_LANES = 16


def _sc_dispatch(h, idx, rank, offs):
    mesh = plsc.VectorSubcoreMesh(core_axis_name="c", subcore_axis_name="s")

    @functools.partial(
        pl.kernel, mesh=mesh,
        out_type=[
            jax.ShapeDtypeStruct((N_TOK, D_H), jnp.float32),
            jax.ShapeDtypeStruct((N_TOK,), jnp.int32),
        ],
        scratch_types=[
            pltpu.VMEM((_CHUNK,), jnp.int32),
            pltpu.VMEM((_CHUNK,), jnp.int32),
            pltpu.VMEM((_CHUNK,), jnp.int32),
            pltpu.VMEM((_CHUNK,), jnp.int32),
            pltpu.VMEM((_CHUNK, D_H), jnp.float32),
            pltpu.SemaphoreType.DMA,
            pltpu.SemaphoreType.DMA,
            pltpu.SemaphoreType.DMA,
        ],
    )
    def k(h_hbm, idx_hbm, rank_hbm, offs_hbm, hs_hbm, pos_hbm,
          idx_v, rank_v, offsg_v, pos_v, rows_v, sem_rows, sem_meta, sem_idx):
        wid = lax.axis_index("s") * _NC + lax.axis_index("c")
        base = wid * _CHUNK
        cp_rows = pltpu.async_copy(h_hbm.at[pl.ds(base, _CHUNK)], rows_v,
                                   sem_rows)
        cp_idx = pltpu.async_copy(idx_hbm.at[pl.ds(base, _CHUNK)], idx_v,
                                  sem_idx)
        cp_rank = pltpu.async_copy(rank_hbm.at[pl.ds(base, _CHUNK)], rank_v,
                                   sem_meta)
        cp_idx.wait()
        cp_off = pltpu.async_copy(offs_hbm.at[idx_v], offsg_v, sem_idx)
        cp_rank.wait()
        cp_off.wait()
        for i in range(_CHUNK // _LANES):
            sl = pl.ds(i * _LANES, _LANES)
            pos_v[sl] = offsg_v[sl] + rank_v[sl]
        cp_rows.wait()
        cp_sc = pltpu.async_copy(rows_v, hs_hbm.at[pos_v], sem_rows)
        pltpu.sync_copy(pos_v, pos_hbm.at[pl.ds(base, _CHUNK)])
        cp_sc.wait()

    return k(h, idx, rank, offs)


def _sc_combine(ys, pos):
    mesh = plsc.VectorSubcoreMesh(core_axis_name="c", subcore_axis_name="s")

    @functools.partial(
        pl.kernel, mesh=mesh,
        out_type=jax.ShapeDtypeStruct((N_TOK, D_H), jnp.float32),
        scratch_types=[
            pltpu.VMEM((_CHUNK,), jnp.int32),
            pltpu.VMEM((_CHUNK, D_H), jnp.float32),
            pltpu.SemaphoreType.DMA,
        ],
    )
    def k(ys_hbm, pos_hbm, out_hbm, pos_v, rows_v, sem):
        wid = lax.axis_index("s") * _NC + lax.axis_index("c")
        base = wid * _CHUNK
        pltpu.sync_copy(pos_hbm.at[pl.ds(base, _CHUNK)], pos_v)
        pltpu.async_copy(ys_hbm.at[pos_v], rows_v, sem).wait()
        pltpu.sync_copy(rows_v, out_hbm.at[pl.ds(base, _CHUNK)])

    return k(ys, pos)


def _stage2_body(offs_ref, cnt_ref, hs_ref, w_ref, b_ref, out_ref):
    e = pl.program_id(0)
    start = offs_ref[e]
    cnt = cnt_ref[e]
    s0 = (start // 8) * 8
    n = (cnt + (start - s0) + TILE - 1) // TILE
    w = w_ref[0]
    b = b_ref[0]

    def body(j, _):
        s = jnp.minimum(s0 + j * TILE, N_TOK - TILE)
        s = pl.multiple_of(s, 8)
        rows = hs_ref[pl.ds(s, TILE), :]
        y = jnp.dot(rows, w, preferred_element_type=jnp.float32) + b
        rid = s + lax.broadcasted_iota(jnp.int32, (TILE, 1), 0)
        msk = (rid >= start) & (rid < start + cnt)
        cur = out_ref[pl.ds(s, TILE), :]
        out_ref[pl.ds(s, TILE), :] = jnp.where(msk, y, cur)
        return 0

    lax.fori_loop(0, n, body, 0)


def _stage2(offs, cnt, hs, w_experts, b_experts):
    return pl.pallas_call(
        _stage2_body,
        grid=(N_EXP,),
        in_specs=[
            pl.BlockSpec(memory_space=pltpu.SMEM),
            pl.BlockSpec(memory_space=pltpu.SMEM),
            pl.BlockSpec((N_TOK, D_H), lambda e: (0, 0)),
            pl.BlockSpec((1, D_H, D_H), lambda e: (e, 0, 0)),
            pl.BlockSpec((1, 1, D_H), lambda e: (e, 0, 0)),
        ],
        out_specs=pl.BlockSpec((N_TOK, D_H), lambda e: (0, 0)),
        out_shape=jax.ShapeDtypeStruct((N_TOK, D_H), jnp.float32),
    )(offs, cnt, hs, w_experts, b_experts)


def _stage3_body(my_ref, g_ref, rc_ref, wd_ref, bd_ref, out_ref):
    mixed = my_ref[...] * g_ref[...] + rc_ref[...]
    out_ref[...] = (
        jnp.dot(mixed, wd_ref[...], preferred_element_type=jnp.float32)
        + bd_ref[...])


def _stage3(my, g, rc, w_dec, b_dec):
    full = lambda shape: pl.BlockSpec(shape, lambda t: (0, 0))
    tok = lambda w: pl.BlockSpec((TILE, w), lambda t: (t, 0))
    return pl.pallas_call(
        _stage3_body,
        grid=(N_TILES,),
        in_specs=[tok(D_H), tok(1), tok(D_H), full((D_H, D_IN)), full((1, D_IN))],
        out_specs=tok(D_IN),
        out_shape=jax.ShapeDtypeStruct((N_TOK, D_IN), jnp.float32),
    )(my, g, rc, w_dec, b_dec)


def kernel(x, W_enc, b_enc, W_gate, W_experts, b_experts, W_res, b_res,
           W_coef, b_coef, W_dec, b_dec):
    h, rc, g, idx, rank, cnt, offs = _stage1(
        x, W_enc, b_enc.reshape(1, D_H), W_gate, W_res, b_res.reshape(1, D_H),
        W_coef, b_coef.reshape(1, 2))
    hs, pos = _sc_dispatch(h, idx.reshape(N_TOK), rank.reshape(N_TOK),
                           offs.reshape(N_EXP))
    ys = _stage2(offs.reshape(N_EXP), cnt.reshape(N_EXP), hs,
                 W_experts, b_experts.reshape(N_EXP, 1, D_H))
    my = _sc_combine(ys, pos)
    return _stage3(my, g, rc, W_dec, b_dec.reshape(1, D_IN))
